# baseline (speedup 1.0000x reference)
import dataclasses
import functools

import jax
import jax.numpy as jnp
from jax import lax
from jax.experimental import pallas as pl
from jax.experimental.pallas import tpu as pltpu
from jax.experimental.pallas import tpu_sc as plsc

L = 16
K = 2
U = 4


def _gather_rows(B, C, N, M):
  info = plsc.get_sparse_core_info()
  NC, NS = info.num_cores, info.num_subcores
  NW = NC * NS
  ROWS = B * C
  assert ROWS % K == 0 and C % K == 0
  SR = ROWS // K
  assert SR % NW == 0
  SPW = SR // NW
  assert SPW % 2 == 0
  assert (C // K) % SPW == 0 or SPW % (C // K) == 0
  assert M % (L * U) == 0

  mesh = plsc.VectorSubcoreMesh(core_axis_name="c", subcore_axis_name="s")

  cp = pltpu.CompilerParams()
  if "needs_layout_passes" in pltpu.CompilerParams.__dataclass_fields__:
    cp = dataclasses.replace(cp, needs_layout_passes=False)

  @functools.partial(
      pl.kernel,
      compiler_params=cp,
      out_type=jax.ShapeDtypeStruct((SR, K * M), jnp.float32),
      mesh=mesh,
      scratch_types=[
          pltpu.VMEM((M,), jnp.int32),
          pltpu.VMEM((K * N,), jnp.float32),
          pltpu.VMEM((K * N,), jnp.float32),
          pltpu.VMEM((K * M,), jnp.float32),
          pltpu.VMEM((K * M,), jnp.float32),
          pltpu.SemaphoreType.DMA((2,)),
          pltpu.SemaphoreType.DMA((2,)),
      ],
  )
  def k(f_hbm, i_hbm, o_hbm, idx_v, row_a, row_b, out_a, out_b,
        sem_in, sem_out):
    wid = lax.axis_index("s") * NC + lax.axis_index("c")
    s0 = wid * SPW
    b = (s0 * K) // C
    rows = (row_a, row_b)
    outs = (out_a, out_b)

    pltpu.sync_copy(i_hbm.at[b], idx_v)

    pltpu.async_copy(f_hbm.at[s0], row_a, sem_in.at[0])
    pltpu.async_copy(f_hbm.at[s0 + 1], row_b, sem_in.at[1])

    @pl.loop(0, SPW, step=2)
    def _(g):
      for p in range(2):
        r = g + p
        pltpu.make_async_copy(f_hbm.at[s0 + r], rows[p],
                              sem_in.at[p]).wait()

        @pl.when(r >= 2)
        def _():
          pltpu.make_async_copy(outs[p], o_hbm.at[s0 + r - 2],
                                sem_out.at[p]).wait()

        @pl.loop(0, M, step=L * U)
        def _(i):
          for u in range(U):
            off = i + u * L
            idxv = idx_v[pl.ds(off, L)]
            for kk in range(K):
              vals = plsc.load_gather(
                  rows[p], [idxv + jnp.int32(kk * N)] if kk else [idxv])
              outs[p][pl.ds(off + kk * M, L)] = vals

        pltpu.async_copy(outs[p], o_hbm.at[s0 + r], sem_out.at[p])

        @pl.when(r + 2 < SPW)
        def _():
          pltpu.async_copy(f_hbm.at[s0 + r + 2], rows[p], sem_in.at[p])

    pltpu.make_async_copy(out_a, o_hbm.at[s0 + SPW - 2],
                          sem_out.at[0]).wait()
    pltpu.make_async_copy(out_b, o_hbm.at[s0 + SPW - 1],
                          sem_out.at[1]).wait()

  return k


@jax.jit
def kernel(features, indices):
  B, C, N = features.shape
  M = indices.shape[1]
  k = _gather_rows(B, C, N, M)
  out = k(features.reshape(B * C // K, K * N), indices)
  return out.reshape(B, C, M)


# speedup vs baseline: 2.8676x; 2.8676x over previous
import dataclasses
import functools

import jax
import jax.numpy as jnp
from jax import lax
from jax.experimental import pallas as pl
from jax.experimental.pallas import tpu as pltpu
from jax.experimental.pallas import tpu_sc as plsc

L = 16
U = 4


def _gather_rows(B, C, N, M):
  info = plsc.get_sparse_core_info()
  NC, NS = info.num_cores, info.num_subcores
  NW = NC * NS
  ROWS = B * C
  assert ROWS % NW == 0
  RPW = ROWS // NW
  assert (C % RPW == 0) or (RPW % C == 0)
  assert M % (L * U) == 0

  mesh = plsc.VectorSubcoreMesh(core_axis_name="c", subcore_axis_name="s")

  cp = pltpu.CompilerParams()
  if "needs_layout_passes" in pltpu.CompilerParams.__dataclass_fields__:
    cp = dataclasses.replace(cp, needs_layout_passes=False)

  @functools.partial(
      pl.kernel,
      compiler_params=cp,
      out_type=jax.ShapeDtypeStruct((ROWS, M), jnp.float32),
      mesh=mesh,
      scratch_types=[
          pltpu.VMEM((M,), jnp.int32),
          pltpu.VMEM((N,), jnp.float32),
          pltpu.VMEM((N,), jnp.float32),
          pltpu.VMEM((M,), jnp.float32),
          pltpu.VMEM((M,), jnp.float32),
          pltpu.SemaphoreType.DMA((2,)),
          pltpu.SemaphoreType.DMA((2,)),
      ],
  )
  def k(f_hbm, i_hbm, o_hbm, idx_v, row_a, row_b, out_a, out_b,
        sem_in, sem_out):
    wid = lax.axis_index("s") * NC + lax.axis_index("c")
    row0 = wid * RPW
    b = row0 // C
    rows = (row_a, row_b)
    outs = (out_a, out_b)

    pltpu.sync_copy(i_hbm.at[b], idx_v)

    pltpu.async_copy(f_hbm.at[row0], row_a, sem_in.at[0])
    pltpu.async_copy(f_hbm.at[row0 + 1], row_b, sem_in.at[1])

    @pl.loop(0, RPW, step=2)
    def _(g):
      for p in range(2):
        r = g + p
        pltpu.make_async_copy(f_hbm.at[row0 + r], rows[p],
                              sem_in.at[p]).wait()

        @pl.when(r >= 2)
        def _():
          pltpu.make_async_copy(outs[p], o_hbm.at[row0 + r - 2],
                                sem_out.at[p]).wait()

        @pl.loop(0, M, step=L * U)
        def _(i):
          for u in range(U):
            off = i + u * L
            idxv = idx_v[pl.ds(off, L)]
            outs[p][pl.ds(off, L)] = plsc.load_gather(rows[p], [idxv])

        pltpu.async_copy(outs[p], o_hbm.at[row0 + r], sem_out.at[p])

        @pl.when(r + 2 < RPW)
        def _():
          pltpu.async_copy(f_hbm.at[row0 + r + 2], rows[p], sem_in.at[p])

    pltpu.make_async_copy(out_a, o_hbm.at[row0 + RPW - 2],
                          sem_out.at[0]).wait()
    pltpu.make_async_copy(out_b, o_hbm.at[row0 + RPW - 1],
                          sem_out.at[1]).wait()

  return k


@jax.jit
def kernel(features, indices):
  B, C, N = features.shape
  M = indices.shape[1]
  k = _gather_rows(B, C, N, M)
  out = k(features.reshape(B * C, N), indices)
  return out.reshape(B, C, M)
